# baseline (speedup 1.0000x reference)
import functools

import jax
import jax.numpy as jnp
from jax import lax
from jax.experimental import pallas as pl
from jax.experimental.pallas import tpu as pltpu
from jax.experimental.pallas import tpu_sc as plsc

BATCH = 16384
USER_DIM = 64
CAT_DIM = 32
MONTH_DIM = 16
HIDDEN = 128

NC = 2
NS = 16
NW = NC * NS
B_PER_W = BATCH // NW
CHUNK = 128
N_CHUNK = B_PER_W // CHUNK

_sc_mesh = plsc.VectorSubcoreMesh(core_axis_name="c", subcore_axis_name="s")


@functools.partial(
    pl.kernel,
    mesh=_sc_mesh,
    out_type=(
        jax.ShapeDtypeStruct((BATCH, 128), jnp.float32),
        jax.ShapeDtypeStruct((BATCH, 128), jnp.float32),
    ),
    scratch_types=[
        pltpu.VMEM((N_CHUNK, CHUNK), jnp.int32),
        pltpu.VMEM((N_CHUNK, CHUNK), jnp.int32),
        pltpu.VMEM((CHUNK, 128), jnp.float32),
        pltpu.VMEM((CHUNK, 128), jnp.float32),
        pltpu.VMEM((CHUNK, 128), jnp.float32),
        pltpu.VMEM((CHUNK, 128), jnp.float32),
        pltpu.SemaphoreType.DMA,
    ],
    compiler_params=pltpu.CompilerParams(use_tc_tiling_on_sc=True),
)
def _sc_gather(uid_hbm, cid_hbm, ut_hbm, ct_hbm,
               u_out, c_out,
               idx_u, idx_c, rows_u0, rows_u1, rows_c0, rows_c1, sem):
    wid = lax.axis_index("s") * NC + lax.axis_index("c")
    base_row = wid * N_CHUNK
    pltpu.sync_copy(uid_hbm.at[pl.ds(base_row, N_CHUNK)], idx_u)
    pltpu.sync_copy(cid_hbm.at[pl.ds(base_row, N_CHUNK)], idx_c)
    bufs_u = (rows_u0, rows_u1)
    bufs_c = (rows_c0, rows_c1)
    base = wid * B_PER_W
    handles = []
    for j in range(N_CHUNK):
        handles.append((
            pltpu.async_copy(ut_hbm.at[idx_u.at[j]], bufs_u[j % 2], sem),
            pltpu.async_copy(ct_hbm.at[idx_c.at[j]], bufs_c[j % 2], sem),
        ))
        if j >= 1:
            hu, hc = handles[j - 1]
            hu.wait()
            hc.wait()
            sl = pl.ds(base + (j - 1) * CHUNK, CHUNK)
            pltpu.sync_copy(bufs_u[(j - 1) % 2], u_out.at[sl])
            pltpu.sync_copy(bufs_c[(j - 1) % 2], c_out.at[sl])
    hu, hc = handles[N_CHUNK - 1]
    hu.wait()
    hc.wait()
    sl = pl.ds(base + (N_CHUNK - 1) * CHUNK, CHUNK)
    pltpu.sync_copy(bufs_u[(N_CHUNK - 1) % 2], u_out.at[sl])
    pltpu.sync_copy(bufs_c[(N_CHUNK - 1) % 2], c_out.at[sl])


BLK = 2048


def _mlp_body(u_ref, c_ref, uid_ref, cid_ref, mid_ref,
              w1u_ref, w1c_ref, mt_ref, w1m_ref, b1_ref,
              w2_ref, b2_ref, o_ref):
    u128 = u_ref[...]
    par = (uid_ref[...] & 1) == 1
    u = jnp.where(par, u128[:, USER_DIM:], u128[:, :USER_DIM])
    c128 = c_ref[...]
    q = cid_ref[...] & 3
    c = jnp.where(
        q < 2,
        jnp.where(q == 0, c128[:, 0:32], c128[:, 32:64]),
        jnp.where(q == 2, c128[:, 64:96], c128[:, 96:128]),
    )
    oh = (mid_ref[...] == lax.broadcasted_iota(jnp.int32, (1, MONTH_DIM), 1))
    oh = oh.astype(jnp.float32)
    mt_w = jnp.dot(mt_ref[...], w1m_ref[...], preferred_element_type=jnp.float32,
                   precision=lax.Precision.HIGHEST)
    h = (
        jnp.dot(u, w1u_ref[...], preferred_element_type=jnp.float32,
                precision=lax.Precision.HIGHEST)
        + jnp.dot(c, w1c_ref[...], preferred_element_type=jnp.float32,
                  precision=lax.Precision.HIGHEST)
        + jnp.dot(oh, mt_w, preferred_element_type=jnp.float32,
                  precision=lax.Precision.HIGHEST)
        + b1_ref[...]
    )
    h = jnp.maximum(h, 0.0)
    o_ref[...] = (
        jnp.dot(h, w2_ref[...], preferred_element_type=jnp.float32,
                precision=lax.Precision.HIGHEST)
        + b2_ref[...]
    )


def _mlp(u_e, c_e, uid2, cid2, mid2, w1u, w1c, mt16, w1m, b1, w2, b2):
    grid = (BATCH // BLK,)
    return pl.pallas_call(
        _mlp_body,
        grid=grid,
        in_specs=[
            pl.BlockSpec((BLK, 128), lambda i: (i, 0)),
            pl.BlockSpec((BLK, 128), lambda i: (i, 0)),
            pl.BlockSpec((BLK, 1), lambda i: (i, 0)),
            pl.BlockSpec((BLK, 1), lambda i: (i, 0)),
            pl.BlockSpec((BLK, 1), lambda i: (i, 0)),
            pl.BlockSpec((USER_DIM, HIDDEN), lambda i: (0, 0)),
            pl.BlockSpec((CAT_DIM, HIDDEN), lambda i: (0, 0)),
            pl.BlockSpec((MONTH_DIM, MONTH_DIM), lambda i: (0, 0)),
            pl.BlockSpec((MONTH_DIM, HIDDEN), lambda i: (0, 0)),
            pl.BlockSpec((1, HIDDEN), lambda i: (0, 0)),
            pl.BlockSpec((HIDDEN, USER_DIM), lambda i: (0, 0)),
            pl.BlockSpec((1, USER_DIM), lambda i: (0, 0)),
        ],
        out_specs=pl.BlockSpec((BLK, USER_DIM), lambda i: (i, 0)),
        out_shape=jax.ShapeDtypeStruct((BATCH, USER_DIM), jnp.float32),
        compiler_params=pltpu.CompilerParams(
            dimension_semantics=("arbitrary",),
        ),
    )(u_e, c_e, uid2, cid2, mid2, w1u, w1c, mt16, w1m, b1, w2, b2)


def kernel(user_id, category_id, month, user_table, cat_table, month_table,
           W1, b1, W2, b2):
    uid = user_id.astype(jnp.int32)
    cid = category_id.astype(jnp.int32)
    mid = month.astype(jnp.int32)
    ut128 = user_table.reshape(USER_VOCAB_HALF, 128)
    ct128 = cat_table.reshape(CAT_VOCAB_QUARTER, 128)
    uid_g = (uid >> 1).reshape(NW * N_CHUNK, CHUNK)
    cid_g = (cid >> 2).reshape(NW * N_CHUNK, CHUNK)
    u_e, c_e = _sc_gather(uid_g, cid_g, ut128, ct128)
    mt16 = jnp.zeros((MONTH_DIM, MONTH_DIM), jnp.float32).at[:12].set(month_table)
    w1t = W1.T
    w1u = w1t[:USER_DIM]
    w1c = w1t[USER_DIM:USER_DIM + CAT_DIM]
    w1m = w1t[USER_DIM + CAT_DIM:]
    return _mlp(u_e, c_e, uid.reshape(BATCH, 1), cid.reshape(BATCH, 1),
                mid.reshape(BATCH, 1), w1u, w1c, mt16, w1m,
                b1.reshape(1, HIDDEN), W2.T, b2.reshape(1, USER_DIM))


USER_VOCAB_HALF = 500000
CAT_VOCAB_QUARTER = 250


# speedup vs baseline: 1.0017x; 1.0017x over previous
import functools

import jax
import jax.numpy as jnp
from jax import lax
from jax.experimental import pallas as pl
from jax.experimental.pallas import tpu as pltpu
from jax.experimental.pallas import tpu_sc as plsc

BATCH = 16384
USER_DIM = 64
CAT_VOCAB = 1000
CAT_DIM = 32
MONTH_DIM = 16
HIDDEN = 128

NC = 2
NS = 16
NW = NC * NS
B_PER_W = BATCH // NW
CHUNK = 128
N_CHUNK = B_PER_W // CHUNK

_sc_mesh = plsc.VectorSubcoreMesh(core_axis_name="c", subcore_axis_name="s")


@functools.partial(
    pl.kernel,
    mesh=_sc_mesh,
    out_type=jax.ShapeDtypeStruct((BATCH, 128), jnp.float32),
    scratch_types=[
        pltpu.VMEM((N_CHUNK, CHUNK), jnp.int32),
        pltpu.VMEM((CHUNK, 128), jnp.float32),
        pltpu.VMEM((CHUNK, 128), jnp.float32),
        pltpu.SemaphoreType.DMA,
    ],
    compiler_params=pltpu.CompilerParams(use_tc_tiling_on_sc=True),
)
def _sc_gather(uid_hbm, ut_hbm, u_out, idx_u, rows0, rows1, sem):
    wid = lax.axis_index("s") * NC + lax.axis_index("c")
    base_row = wid * N_CHUNK
    pltpu.sync_copy(uid_hbm.at[pl.ds(base_row, N_CHUNK)], idx_u)
    bufs = (rows0, rows1)
    base = wid * B_PER_W
    handles = []
    for j in range(N_CHUNK):
        handles.append(pltpu.async_copy(ut_hbm.at[idx_u.at[j]], bufs[j % 2], sem))
        if j >= 1:
            handles[j - 1].wait()
            pltpu.sync_copy(bufs[(j - 1) % 2],
                            u_out.at[pl.ds(base + (j - 1) * CHUNK, CHUNK)])
    handles[N_CHUNK - 1].wait()
    pltpu.sync_copy(bufs[(N_CHUNK - 1) % 2],
                    u_out.at[pl.ds(base + (N_CHUNK - 1) * CHUNK, CHUNK)])


BLK = 2048


def _mlp_body(u_ref, cid_ref, mid_ref, w1u_ref, w1c_ref, catT_ref,
              mt_ref, w1m_ref, b1_ref, w2_ref, b2_ref, o_ref):
    hp = lax.Precision.HIGHEST
    u = u_ref[...][:, :USER_DIM]
    u_h = jnp.dot(u, w1u_ref[...], precision=hp,
                  preferred_element_type=jnp.float32)
    oh_c = (cid_ref[...] == lax.broadcasted_iota(jnp.int32, (1, CAT_VOCAB), 1))
    c_emb = lax.dot_general(oh_c.astype(jnp.float32), catT_ref[...],
                            (((1,), (1,)), ((), ())),
                            precision=hp, preferred_element_type=jnp.float32)
    c_h = jnp.dot(c_emb, w1c_ref[...], precision=hp,
                  preferred_element_type=jnp.float32)
    oh_m = (mid_ref[...] == lax.broadcasted_iota(jnp.int32, (1, MONTH_DIM), 1))
    mt_w = jnp.dot(mt_ref[...], w1m_ref[...], precision=hp,
                   preferred_element_type=jnp.float32)
    m_h = jnp.dot(oh_m.astype(jnp.float32), mt_w, precision=hp,
                  preferred_element_type=jnp.float32)
    h = jnp.maximum(u_h + c_h + m_h + b1_ref[...], 0.0)
    o_ref[...] = (
        jnp.dot(h, w2_ref[...], precision=hp, preferred_element_type=jnp.float32)
        + b2_ref[...]
    )


def _mlp(u_e, cid2, mid2, w1u, w1c, catT, mt16, w1m, b1, w2, b2):
    grid = (BATCH // BLK,)
    return pl.pallas_call(
        _mlp_body,
        grid=grid,
        in_specs=[
            pl.BlockSpec((BLK, 128), lambda i: (i, 0)),
            pl.BlockSpec((BLK, 1), lambda i: (i, 0)),
            pl.BlockSpec((BLK, 1), lambda i: (i, 0)),
            pl.BlockSpec((USER_DIM, HIDDEN), lambda i: (0, 0)),
            pl.BlockSpec((CAT_DIM, HIDDEN), lambda i: (0, 0)),
            pl.BlockSpec((CAT_DIM, CAT_VOCAB), lambda i: (0, 0)),
            pl.BlockSpec((MONTH_DIM, MONTH_DIM), lambda i: (0, 0)),
            pl.BlockSpec((MONTH_DIM, HIDDEN), lambda i: (0, 0)),
            pl.BlockSpec((1, HIDDEN), lambda i: (0, 0)),
            pl.BlockSpec((HIDDEN, USER_DIM), lambda i: (0, 0)),
            pl.BlockSpec((1, USER_DIM), lambda i: (0, 0)),
        ],
        out_specs=pl.BlockSpec((BLK, USER_DIM), lambda i: (i, 0)),
        out_shape=jax.ShapeDtypeStruct((BATCH, USER_DIM), jnp.float32),
        compiler_params=pltpu.CompilerParams(
            dimension_semantics=("arbitrary",),
        ),
    )(u_e, cid2, mid2, w1u, w1c, catT, mt16, w1m, b1, w2, b2)


def kernel(user_id, category_id, month, user_table, cat_table, month_table,
           W1, b1, W2, b2):
    uid = user_id.astype(jnp.int32)
    cid = category_id.astype(jnp.int32)
    mid = month.astype(jnp.int32)
    ut128 = lax.pad(user_table, jnp.float32(0), ((0, 0, 0), (0, 64, 0)))
    catT = cat_table.T
    u_e = _sc_gather(uid.reshape(NW * N_CHUNK, CHUNK), ut128)
    mt16 = jnp.zeros((MONTH_DIM, MONTH_DIM), jnp.float32).at[:12].set(month_table)
    w1t = W1.T
    w1u = w1t[:USER_DIM]
    w1c = w1t[USER_DIM:USER_DIM + CAT_DIM]
    w1m = w1t[USER_DIM + CAT_DIM:]
    return _mlp(u_e, cid.reshape(BATCH, 1), mid.reshape(BATCH, 1),
                w1u, w1c, catT, mt16, w1m,
                b1.reshape(1, HIDDEN), W2.T, b2.reshape(1, USER_DIM))
